# Initial kernel scaffold; baseline (speedup 1.0000x reference)
import jax
import jax.numpy as jnp
from jax.experimental import pallas as pl


def kernel(tree_tensor, filler_emb, role_emb):
    raise NotImplementedError("write your pallas kernel here")


# speedup vs baseline: 3.2202x; 3.2202x over previous
import functools

import jax
import jax.numpy as jnp
from jax import lax
from jax.experimental import pallas as pl
from jax.experimental.pallas import tpu as pltpu
from jax.experimental.pallas import tpu_sc as plsc

B = 4096
R = 200
DF = 128
DR = 128
NB = B * R

NC = 2
NS = 16
NW = NC * NS
ROWS_PER_W = NB // NW
CHUNK = 128
N_CHUNKS = ROWS_PER_W // CHUNK


def _gather_sc(filler_emb, idx_flat):
    mesh = plsc.VectorSubcoreMesh(core_axis_name="c", subcore_axis_name="s")

    @functools.partial(
        pl.kernel,
        mesh=mesh,
        out_type=jax.ShapeDtypeStruct((NB, DF), filler_emb.dtype),
        scratch_types=[
            pltpu.VMEM((ROWS_PER_W,), jnp.int32),
            pltpu.VMEM((2, CHUNK, DF), filler_emb.dtype),
            pltpu.SemaphoreType.DMA,
        ],
    )
    def k(table_hbm, idx_hbm, out_hbm, idx_v, rows_v, gsem):
        wid = lax.axis_index("s") * NC + lax.axis_index("c")
        base = wid * ROWS_PER_W
        pltpu.sync_copy(idx_hbm.at[pl.ds(base, ROWS_PER_W)], idx_v)

        def body(j, carry):
            g = pltpu.async_copy(
                table_hbm.at[idx_v.at[pl.ds(j * CHUNK, CHUNK)]],
                rows_v.at[0],
                gsem,
            )
            g.wait()
            pltpu.sync_copy(rows_v.at[0], out_hbm.at[pl.ds(base + j * CHUNK, CHUNK)])
            return carry

        lax.fori_loop(0, N_CHUNKS, body, 0)

    return k(filler_emb, idx_flat)


BB = 8


def _mm_body(x_ref, role_ref, out_ref):
    for i in range(BB):
        out_ref[i] = lax.dot_general(
            x_ref[i],
            role_ref[...],
            (((0,), (0,)), ((), ())),
            preferred_element_type=jnp.float32,
        )


def _tpr_tc(x, role_emb):
    return pl.pallas_call(
        _mm_body,
        grid=(B // BB,),
        in_specs=[
            pl.BlockSpec((BB, R, DF), lambda i: (i, 0, 0)),
            pl.BlockSpec((R, DR), lambda i: (0, 0)),
        ],
        out_specs=pl.BlockSpec((BB, DF, DR), lambda i: (i, 0, 0)),
        out_shape=jax.ShapeDtypeStruct((B, DF, DR), jnp.float32),
    )(x, role_emb)


def kernel(tree_tensor, filler_emb, role_emb):
    idx_flat = tree_tensor.reshape(-1)
    x = _gather_sc(filler_emb, idx_flat)
    return _tpr_tc(x.reshape(B, R, DF), role_emb)
